# baseline (speedup 1.0000x reference)
import functools
import math

import jax
import jax.numpy as jnp
from jax.experimental import pallas as pl

B, S, D = 1, 2048, 1024
H, DH = 16, 64
E, TOPK, FF = 8, 2, 2048
EPS = 1e-6

S_TILE = 256
NT = S // S_TILE
MOE_TILE = 512
NMT = S // MOE_TILE

_NEG = -1e30
_LN1E4 = math.log(10000.0)


def _rms(h, w):
    var = jnp.mean(h * h, axis=-1, keepdims=True)
    return h * jax.lax.rsqrt(var + EPS) * w


def _k1_body(x_ref, ln1_ref, wql_ref, wqh_ref, wkl_ref, wkh_ref, wv_ref,
             qa_ref, qb_ref, ka_ref, kb_ref, v_ref):
    i = pl.program_id(0)
    h = _rms(x_ref[...], ln1_ref[...])
    ql = jnp.dot(h, wql_ref[...], preferred_element_type=jnp.float32)
    qh = jnp.dot(h, wqh_ref[...], preferred_element_type=jnp.float32)
    kl = jnp.dot(h, wkl_ref[...], preferred_element_type=jnp.float32)
    kh = jnp.dot(h, wkh_ref[...], preferred_element_type=jnp.float32)
    v_ref[...] = jnp.dot(h, wv_ref[...], preferred_element_type=jnp.float32)
    pos = (i * S_TILE + jax.lax.broadcasted_iota(jnp.int32, (S_TILE, H * DH // 2), 0)
           ).astype(jnp.float32)
    lane = jax.lax.broadcasted_iota(jnp.int32, (S_TILE, H * DH // 2), 1) % (DH // 2)
    inv_freq = jnp.exp(lane.astype(jnp.float32) * (-2.0 * _LN1E4 / DH))
    theta = pos * inv_freq
    c = jnp.cos(theta)
    s = jnp.sin(theta)
    qa_ref[...] = ql * c - qh * s
    qb_ref[...] = qh * c + ql * s
    ka_ref[...] = kl * c - kh * s
    kb_ref[...] = kh * c + kl * s


def _k2_body(qa_ref, qb_ref, ka_ref, kb_ref, v_ref, o_ref):
    qi = pl.program_id(0)
    rowp = qi * S_TILE + jax.lax.broadcasted_iota(jnp.int32, (S_TILE, S_TILE), 0)
    scale = 1.0 / math.sqrt(DH)
    for h in range(H):
        ha = slice(h * (DH // 2), (h + 1) * (DH // 2))
        hv = slice(h * DH, (h + 1) * DH)
        q = jnp.concatenate([qa_ref[:, ha], qb_ref[:, ha]], axis=1) * scale

        def body(j, carry, q=q):
            m, l, acc = carry
            k = jnp.concatenate(
                [ka_ref[pl.ds(j * S_TILE, S_TILE), ha],
                 kb_ref[pl.ds(j * S_TILE, S_TILE), ha]], axis=1)
            v = v_ref[pl.ds(j * S_TILE, S_TILE), hv]
            s = jax.lax.dot_general(q, k, (((1,), (1,)), ((), ())),
                                    preferred_element_type=jnp.float32)
            colp = j * S_TILE + jax.lax.broadcasted_iota(
                jnp.int32, (S_TILE, S_TILE), 1)
            s = jnp.where(colp <= rowp, s, _NEG)
            m_new = jnp.maximum(m, jnp.max(s, axis=1, keepdims=True))
            p = jnp.exp(s - m_new)
            sc = jnp.exp(m - m_new)
            l_new = l * sc + jnp.sum(p, axis=1, keepdims=True)
            acc_new = acc * sc + jnp.dot(p, v, preferred_element_type=jnp.float32)
            return m_new, l_new, acc_new

        m0 = jnp.full((S_TILE, 1), _NEG, jnp.float32)
        l0 = jnp.zeros((S_TILE, 1), jnp.float32)
        a0 = jnp.zeros((S_TILE, DH), jnp.float32)
        m, l, acc = jax.lax.fori_loop(0, qi + 1, body, (m0, l0, a0))
        o_ref[:, hv] = acc / l


def _k3_body(ctx_ref, x_ref, wo_ref, ln2_ref, rw_ref, x1_ref, h2_ref, lg_ref):
    x1 = x_ref[...] + jnp.dot(ctx_ref[...], wo_ref[...],
                              preferred_element_type=jnp.float32)
    x1_ref[...] = x1
    h2 = _rms(x1, ln2_ref[...])
    h2_ref[...] = h2
    lg_ref[...] = jnp.dot(h2, rw_ref[...], preferred_element_type=jnp.float32)


def _k4_body(lg_ref, gates_ref, aux_ref):
    lg = lg_ref[...]
    mx = jnp.max(lg, axis=1, keepdims=True)
    ex = jnp.exp(lg - mx)
    probs = ex / jnp.sum(ex, axis=1, keepdims=True)
    lane = jax.lax.broadcasted_iota(jnp.int32, (S, E), 1)
    v1 = jnp.max(probs, axis=1, keepdims=True)
    i1 = jnp.min(jnp.where(probs == v1, lane, E), axis=1, keepdims=True)
    oh1 = (lane == i1).astype(jnp.float32)
    masked = jnp.where(lane == i1, _NEG, probs)
    v2 = jnp.max(masked, axis=1, keepdims=True)
    i2 = jnp.min(jnp.where(masked == v2, lane, E), axis=1, keepdims=True)
    oh2 = (lane == i2).astype(jnp.float32)
    tot = v1 + v2
    gates_ref[...] = oh1 * (v1 / tot) + oh2 * (v2 / tot)
    f = jnp.sum(oh1 + oh2, axis=0, keepdims=True) / (S * TOPK)
    pbar = jnp.sum(probs, axis=0, keepdims=True) / S
    aux_ref[...] = E * jnp.sum(f * pbar, axis=1, keepdims=True)


def _k5_body(h2_ref, x1_ref, gates_ref, w1_ref, w2_ref, out_ref):
    e = pl.program_id(1)
    lane = jax.lax.broadcasted_iota(jnp.int32, (MOE_TILE, E), 1)
    ge = jnp.sum(jnp.where(lane == e, gates_ref[...], 0.0), axis=1, keepdims=True)
    a = jnp.dot(h2_ref[...].astype(jnp.bfloat16), w1_ref[0].astype(jnp.bfloat16),
                preferred_element_type=jnp.float32)
    g = jax.nn.gelu(a)
    y = jnp.dot(g.astype(jnp.bfloat16), w2_ref[0].astype(jnp.bfloat16),
                preferred_element_type=jnp.float32)
    contrib = ge * y

    @pl.when(e == 0)
    def _():
        out_ref[...] = x1_ref[...] + contrib

    @pl.when(e != 0)
    def _():
        out_ref[...] = out_ref[...] + contrib


def kernel(x, ln1_w, ln2_w, wq, wk, wv, wo, router_w, w1, w2):
    xs = x.reshape(S, D)
    ln1 = ln1_w.reshape(1, D)
    ln2 = ln2_w.reshape(1, D)
    wq4 = wq.reshape(D, H, 2, DH // 2).transpose(0, 2, 1, 3).reshape(D, 2, D // 2)
    wk4 = wk.reshape(D, H, 2, DH // 2).transpose(0, 2, 1, 3).reshape(D, 2, D // 2)
    wql, wqh = wq4[:, 0], wq4[:, 1]
    wkl, wkh = wk4[:, 0], wk4[:, 1]

    full = lambda shape: pl.BlockSpec(shape, lambda *idx: tuple(0 for _ in shape))
    row_tile = lambda w, tile: pl.BlockSpec((tile, w), lambda i, *_: (i, 0))

    qa, qb, ka, kb, v = pl.pallas_call(
        _k1_body,
        grid=(NT,),
        in_specs=[row_tile(D, S_TILE), full((1, D)), full((D, D // 2)),
                  full((D, D // 2)), full((D, D // 2)), full((D, D // 2)),
                  full((D, D))],
        out_specs=[row_tile(D // 2, S_TILE)] * 4 + [row_tile(D, S_TILE)],
        out_shape=[jax.ShapeDtypeStruct((S, D // 2), jnp.float32)] * 4
        + [jax.ShapeDtypeStruct((S, D), jnp.float32)],
    )(xs, ln1, wql, wqh, wkl, wkh, wv)

    ctx = pl.pallas_call(
        _k2_body,
        grid=(NT,),
        in_specs=[
            row_tile(D // 2, S_TILE),
            row_tile(D // 2, S_TILE),
            full((S, D // 2)),
            full((S, D // 2)),
            full((S, D)),
        ],
        out_specs=row_tile(D, S_TILE),
        out_shape=jax.ShapeDtypeStruct((S, D), jnp.float32),
    )(qa, qb, ka, kb, v)

    x1, h2, logits = pl.pallas_call(
        _k3_body,
        grid=(NT,),
        in_specs=[row_tile(D, S_TILE), row_tile(D, S_TILE), full((D, D)),
                  full((1, D)), full((D, E))],
        out_specs=[row_tile(D, S_TILE), row_tile(D, S_TILE), row_tile(E, S_TILE)],
        out_shape=[jax.ShapeDtypeStruct((S, D), jnp.float32),
                   jax.ShapeDtypeStruct((S, D), jnp.float32),
                   jax.ShapeDtypeStruct((S, E), jnp.float32)],
    )(ctx, xs, wo, ln2, router_w)

    gates, aux = pl.pallas_call(
        _k4_body,
        grid=(1,),
        in_specs=[full((S, E))],
        out_specs=[full((S, E)), full((1, 1))],
        out_shape=[jax.ShapeDtypeStruct((S, E), jnp.float32),
                   jax.ShapeDtypeStruct((1, 1), jnp.float32)],
    )(logits)

    x2 = pl.pallas_call(
        _k5_body,
        grid=(NMT, E),
        in_specs=[
            pl.BlockSpec((MOE_TILE, D), lambda t, e: (t, 0)),
            pl.BlockSpec((MOE_TILE, D), lambda t, e: (t, 0)),
            pl.BlockSpec((MOE_TILE, E), lambda t, e: (t, 0)),
            pl.BlockSpec((1, D, FF), lambda t, e: (e, 0, 0)),
            pl.BlockSpec((1, FF, D), lambda t, e: (e, 0, 0)),
        ],
        out_specs=pl.BlockSpec((MOE_TILE, D), lambda t, e: (t, 0)),
        out_shape=jax.ShapeDtypeStruct((S, D), jnp.float32),
    )(h2, x1, gates, w1, w2)

    return x2.reshape(B, S, D), aux.reshape(())


# speedup vs baseline: 1.0239x; 1.0239x over previous
import functools
import math

import jax
import jax.numpy as jnp
from jax.experimental import pallas as pl
from jax.experimental.pallas import tpu as pltpu
from jax.experimental.pallas import tpu_sc as plsc

B, S, D = 1, 2048, 1024
H, DH = 16, 64
E, TOPK, FF = 8, 2, 2048
EPS = 1e-6

S_TILE = 256
NT = S // S_TILE
BLK = 128
CAP = S * TOPK + E * BLK - E * 1
CAP = 5120
NBLK = CAP // BLK

NW = 32
CHUNK = S // NW

_NEG = -1e30
_LN1E4 = math.log(10000.0)


def _rms(h, w):
    var = jnp.mean(h * h, axis=-1, keepdims=True)
    return h * jax.lax.rsqrt(var + EPS) * w


def _k1_body(x_ref, ln1_ref, wql_ref, wqh_ref, wkl_ref, wkh_ref, wv_ref,
             qa_ref, qb_ref, ka_ref, kb_ref, v_ref):
    i = pl.program_id(0)
    h = _rms(x_ref[...], ln1_ref[...])
    ql = jnp.dot(h, wql_ref[...], preferred_element_type=jnp.float32)
    qh = jnp.dot(h, wqh_ref[...], preferred_element_type=jnp.float32)
    kl = jnp.dot(h, wkl_ref[...], preferred_element_type=jnp.float32)
    kh = jnp.dot(h, wkh_ref[...], preferred_element_type=jnp.float32)
    v_ref[...] = jnp.dot(h, wv_ref[...], preferred_element_type=jnp.float32)
    pos = (i * S_TILE + jax.lax.broadcasted_iota(jnp.int32, (S_TILE, H * DH // 2), 0)
           ).astype(jnp.float32)
    lane = jax.lax.broadcasted_iota(jnp.int32, (S_TILE, H * DH // 2), 1) % (DH // 2)
    inv_freq = jnp.exp(lane.astype(jnp.float32) * (-2.0 * _LN1E4 / DH))
    theta = pos * inv_freq
    c = jnp.cos(theta)
    s = jnp.sin(theta)
    qa_ref[...] = ql * c - qh * s
    qb_ref[...] = qh * c + ql * s
    ka_ref[...] = kl * c - kh * s
    kb_ref[...] = kh * c + kl * s


def _k2_body(qa_ref, qb_ref, ka_ref, kb_ref, v_ref, o_ref):
    qi = pl.program_id(0)
    rowp = qi * S_TILE + jax.lax.broadcasted_iota(jnp.int32, (S_TILE, S_TILE), 0)
    scale = 1.0 / math.sqrt(DH)
    for h in range(H):
        ha = slice(h * (DH // 2), (h + 1) * (DH // 2))
        hv = slice(h * DH, (h + 1) * DH)
        q = jnp.concatenate([qa_ref[:, ha], qb_ref[:, ha]], axis=1) * scale

        def body(j, carry, q=q):
            m, l, acc = carry
            k = jnp.concatenate(
                [ka_ref[pl.ds(j * S_TILE, S_TILE), ha],
                 kb_ref[pl.ds(j * S_TILE, S_TILE), ha]], axis=1)
            v = v_ref[pl.ds(j * S_TILE, S_TILE), hv]
            s = jax.lax.dot_general(q, k, (((1,), (1,)), ((), ())),
                                    preferred_element_type=jnp.float32)
            colp = j * S_TILE + jax.lax.broadcasted_iota(
                jnp.int32, (S_TILE, S_TILE), 1)
            s = jnp.where(colp <= rowp, s, _NEG)
            m_new = jnp.maximum(m, jnp.max(s, axis=1, keepdims=True))
            p = jnp.exp(s - m_new)
            sc = jnp.exp(m - m_new)
            l_new = l * sc + jnp.sum(p, axis=1, keepdims=True)
            acc_new = acc * sc + jnp.dot(p, v, preferred_element_type=jnp.float32)
            return m_new, l_new, acc_new

        m0 = jnp.full((S_TILE, 1), _NEG, jnp.float32)
        l0 = jnp.zeros((S_TILE, 1), jnp.float32)
        a0 = jnp.zeros((S_TILE, DH), jnp.float32)
        m, l, acc = jax.lax.fori_loop(0, qi + 1, body, (m0, l0, a0))
        o_ref[:, hv] = acc / l


def _k3_body(ctx_ref, x_ref, wo_ref, ln2_ref, rw_ref, x1_ref, h2_ref, lg_ref):
    x1 = x_ref[...] + jnp.dot(ctx_ref[...], wo_ref[...],
                              preferred_element_type=jnp.float32)
    x1_ref[...] = x1
    h2 = _rms(x1, ln2_ref[...])
    h2_ref[...] = h2
    lg_ref[...] = jnp.dot(h2, rw_ref[...], preferred_element_type=jnp.float32)


def _k4_body(lg_ref, pos0_ref, pos1_ref, g0_ref, g1_ref, be_ref, aux_ref):
    lg = lg_ref[...]
    mx = jnp.max(lg, axis=1, keepdims=True)
    ex = jnp.exp(lg - mx)
    probs = ex / jnp.sum(ex, axis=1, keepdims=True)
    lane = jax.lax.broadcasted_iota(jnp.int32, (S, E), 1)
    v1 = jnp.max(probs, axis=1, keepdims=True)
    i1 = jnp.min(jnp.where(probs == v1, lane, E), axis=1, keepdims=True)
    oh1 = (lane == i1).astype(jnp.float32)
    masked = jnp.where(lane == i1, _NEG, probs)
    v2 = jnp.max(masked, axis=1, keepdims=True)
    i2 = jnp.min(jnp.where(masked == v2, lane, E), axis=1, keepdims=True)
    oh2 = (lane == i2).astype(jnp.float32)
    tot = v1 + v2
    g0_ref[...] = v1 / tot
    g1_ref[...] = v2 / tot
    m = oh1 + oh2
    lt = (jax.lax.broadcasted_iota(jnp.int32, (S_TILE, S_TILE), 0)
          > jax.lax.broadcasted_iota(jnp.int32, (S_TILE, S_TILE), 1)
          ).astype(jnp.float32)
    rank_chunks = []
    running = jnp.zeros((1, E), jnp.float32)
    for ci in range(NT):
        mc = m[ci * S_TILE:(ci + 1) * S_TILE]
        rank_chunks.append(
            jnp.dot(lt, mc, preferred_element_type=jnp.float32) + running)
        running = running + jnp.sum(mc, axis=0, keepdims=True)
    rank = jnp.concatenate(rank_chunks, axis=0)
    cnt = running
    c_pad = jnp.ceil(cnt * (1.0 / BLK)) * BLK
    up = (jax.lax.broadcasted_iota(jnp.int32, (E, E), 0)
          < jax.lax.broadcasted_iota(jnp.int32, (E, E), 1)).astype(jnp.float32)
    start = jnp.dot(c_pad, up, preferred_element_type=jnp.float32)
    slot = start + rank
    pos0_ref[...] = jnp.sum(oh1 * slot, axis=1, keepdims=True).astype(jnp.int32)
    pos1_ref[...] = jnp.sum(oh2 * slot, axis=1, keepdims=True).astype(jnp.int32)
    bstart = (jax.lax.broadcasted_iota(jnp.int32, (NBLK, E), 0) * BLK
              ).astype(jnp.float32)
    be = jnp.sum((jnp.broadcast_to(start, (NBLK, E)) <= bstart)
                 .astype(jnp.float32), axis=1, keepdims=True) - 1.0
    be_ref[...] = jnp.clip(be, 0.0, E - 1.0).astype(jnp.int32)
    f = jnp.sum(m, axis=0, keepdims=True) / (S * TOPK)
    pbar = jnp.sum(probs, axis=0, keepdims=True) / S
    aux_ref[...] = E * jnp.sum(f * pbar, axis=1, keepdims=True)


def _k5_sc(h2_hbm, pos0_hbm, pos1_hbm, disp_hbm, rows_v, idx_v, sem):
    c = jax.lax.axis_index("c")
    s = jax.lax.axis_index("s")
    wid = s * 2 + c
    base = wid * CHUNK
    pltpu.sync_copy(h2_hbm.at[pl.ds(base, CHUNK), :], rows_v)
    pltpu.sync_copy(pos0_hbm.at[pl.ds(base, CHUNK)], idx_v)
    pltpu.async_copy(rows_v, disp_hbm.at[idx_v], sem).wait()
    pltpu.sync_copy(pos1_hbm.at[pl.ds(base, CHUNK)], idx_v)
    pltpu.async_copy(rows_v, disp_hbm.at[idx_v], sem).wait()


def _k6_body(be_ref, disp_ref, w1_ref, w2_ref, y_ref):
    t = disp_ref[...].astype(jnp.bfloat16)
    a = jnp.dot(t, w1_ref[0].astype(jnp.bfloat16),
                preferred_element_type=jnp.float32)
    g = jax.nn.gelu(a)
    y_ref[...] = jnp.dot(g.astype(jnp.bfloat16), w2_ref[0].astype(jnp.bfloat16),
                         preferred_element_type=jnp.float32)


def _k7_sc(y_hbm, pos0_hbm, pos1_hbm, y0_hbm, y1_hbm, rows_v, idx_v, sem):
    c = jax.lax.axis_index("c")
    s = jax.lax.axis_index("s")
    wid = s * 2 + c
    base = wid * CHUNK
    pltpu.sync_copy(pos0_hbm.at[pl.ds(base, CHUNK)], idx_v)
    pltpu.async_copy(y_hbm.at[idx_v], rows_v, sem).wait()
    pltpu.sync_copy(rows_v, y0_hbm.at[pl.ds(base, CHUNK), :])
    pltpu.sync_copy(pos1_hbm.at[pl.ds(base, CHUNK)], idx_v)
    pltpu.async_copy(y_hbm.at[idx_v], rows_v, sem).wait()
    pltpu.sync_copy(rows_v, y1_hbm.at[pl.ds(base, CHUNK), :])


def _k8_body(x1_ref, y0_ref, y1_ref, g0_ref, g1_ref, out_ref):
    out_ref[...] = (x1_ref[...] + g0_ref[...] * y0_ref[...]
                    + g1_ref[...] * y1_ref[...])


_SC_MESH = dict(core_axis_name="c", subcore_axis_name="s")


def _dispatch(h2, pos0f, pos1f):
    mesh = plsc.VectorSubcoreMesh(**_SC_MESH)
    return pl.kernel(
        _k5_sc,
        mesh=mesh,
        out_type=jax.ShapeDtypeStruct((CAP, D), jnp.float32),
        scratch_types=[pltpu.VMEM((CHUNK, D), jnp.float32),
                       pltpu.VMEM((CHUNK,), jnp.int32),
                       pltpu.SemaphoreType.DMA],
    )(h2, pos0f, pos1f)


def _undispatch(y, pos0f, pos1f):
    mesh = plsc.VectorSubcoreMesh(**_SC_MESH)
    return pl.kernel(
        _k7_sc,
        mesh=mesh,
        out_type=[jax.ShapeDtypeStruct((S, D), jnp.float32),
                  jax.ShapeDtypeStruct((S, D), jnp.float32)],
        scratch_types=[pltpu.VMEM((CHUNK, D), jnp.float32),
                       pltpu.VMEM((CHUNK,), jnp.int32),
                       pltpu.SemaphoreType.DMA],
    )(y, pos0f, pos1f)


def kernel(x, ln1_w, ln2_w, wq, wk, wv, wo, router_w, w1, w2):
    xs = x.reshape(S, D)
    ln1 = ln1_w.reshape(1, D)
    ln2 = ln2_w.reshape(1, D)
    wq4 = wq.reshape(D, H, 2, DH // 2).transpose(0, 2, 1, 3).reshape(D, 2, D // 2)
    wk4 = wk.reshape(D, H, 2, DH // 2).transpose(0, 2, 1, 3).reshape(D, 2, D // 2)
    wql, wqh = wq4[:, 0], wq4[:, 1]
    wkl, wkh = wk4[:, 0], wk4[:, 1]

    full = lambda shape: pl.BlockSpec(shape, lambda *idx: tuple(0 for _ in shape))
    row_tile = lambda w, tile: pl.BlockSpec((tile, w), lambda i, *_: (i, 0))

    qa, qb, ka, kb, v = pl.pallas_call(
        _k1_body,
        grid=(NT,),
        in_specs=[row_tile(D, S_TILE), full((1, D)), full((D, D // 2)),
                  full((D, D // 2)), full((D, D // 2)), full((D, D // 2)),
                  full((D, D))],
        out_specs=[row_tile(D // 2, S_TILE)] * 4 + [row_tile(D, S_TILE)],
        out_shape=[jax.ShapeDtypeStruct((S, D // 2), jnp.float32)] * 4
        + [jax.ShapeDtypeStruct((S, D), jnp.float32)],
    )(xs, ln1, wql, wqh, wkl, wkh, wv)

    ctx = pl.pallas_call(
        _k2_body,
        grid=(NT,),
        in_specs=[row_tile(D // 2, S_TILE), row_tile(D // 2, S_TILE),
                  full((S, D // 2)), full((S, D // 2)), full((S, D))],
        out_specs=row_tile(D, S_TILE),
        out_shape=jax.ShapeDtypeStruct((S, D), jnp.float32),
    )(qa, qb, ka, kb, v)

    x1, h2, logits = pl.pallas_call(
        _k3_body,
        grid=(NT,),
        in_specs=[row_tile(D, S_TILE), row_tile(D, S_TILE), full((D, D)),
                  full((1, D)), full((D, E))],
        out_specs=[row_tile(D, S_TILE), row_tile(D, S_TILE), row_tile(E, S_TILE)],
        out_shape=[jax.ShapeDtypeStruct((S, D), jnp.float32),
                   jax.ShapeDtypeStruct((S, D), jnp.float32),
                   jax.ShapeDtypeStruct((S, E), jnp.float32)],
    )(ctx, xs, wo, ln2, router_w)

    pos0, pos1, g0, g1, be, aux = pl.pallas_call(
        _k4_body,
        grid=(1,),
        in_specs=[full((S, E))],
        out_specs=[full((S, 1)), full((S, 1)), full((S, 1)), full((S, 1)),
                   full((NBLK, 1)), full((1, 1))],
        out_shape=[jax.ShapeDtypeStruct((S, 1), jnp.int32),
                   jax.ShapeDtypeStruct((S, 1), jnp.int32),
                   jax.ShapeDtypeStruct((S, 1), jnp.float32),
                   jax.ShapeDtypeStruct((S, 1), jnp.float32),
                   jax.ShapeDtypeStruct((NBLK, 1), jnp.int32),
                   jax.ShapeDtypeStruct((1, 1), jnp.float32)],
    )(logits)

    pos0f = pos0.reshape(S)
    pos1f = pos1.reshape(S)

    disp = _dispatch(h2, pos0f, pos1f)

    y = pl.pallas_call(
        _k6_body,
        grid_spec=pltpu.PrefetchScalarGridSpec(
            num_scalar_prefetch=1,
            grid=(NBLK,),
            in_specs=[
                pl.BlockSpec((BLK, D), lambda b, be_s: (b, 0)),
                pl.BlockSpec((1, D, FF), lambda b, be_s: (be_s[b], 0, 0)),
                pl.BlockSpec((1, FF, D), lambda b, be_s: (be_s[b], 0, 0)),
            ],
            out_specs=pl.BlockSpec((BLK, D), lambda b, be_s: (b, 0)),
        ),
        out_shape=jax.ShapeDtypeStruct((CAP, D), jnp.float32),
    )(be.reshape(NBLK), disp, w1, w2)

    y0, y1 = _undispatch(y, pos0f, pos1f)

    x2 = pl.pallas_call(
        _k8_body,
        grid=(NT,),
        in_specs=[row_tile(D, S_TILE), row_tile(D, S_TILE), row_tile(D, S_TILE),
                  row_tile(1, S_TILE), row_tile(1, S_TILE)],
        out_specs=row_tile(D, S_TILE),
        out_shape=jax.ShapeDtypeStruct((S, D), jnp.float32),
    )(x1, y0, y1, g0, g1)

    return x2.reshape(B, S, D), aux.reshape(())


# speedup vs baseline: 1.0256x; 1.0016x over previous
import functools
import math

import jax
import jax.numpy as jnp
from jax.experimental import pallas as pl
from jax.experimental.pallas import tpu as pltpu
from jax.experimental.pallas import tpu_sc as plsc

B, S, D = 1, 2048, 1024
H, DH = 16, 64
E, TOPK, FF = 8, 2, 2048
EPS = 1e-6

S_TILE = 256
NT = S // S_TILE
BLK = 128
CAP = S * TOPK + E * BLK - E * 1
CAP = 5120
NBLK = CAP // BLK

NW = 32
CHUNK = S // NW

_NEG = -1e30
_LN1E4 = math.log(10000.0)


def _rms(h, w):
    var = jnp.mean(h * h, axis=-1, keepdims=True)
    return h * jax.lax.rsqrt(var + EPS) * w


def _k1_body(x_ref, ln1_ref, wql_ref, wqh_ref, wkl_ref, wkh_ref, wv_ref,
             qa_ref, qb_ref, ka_ref, kb_ref, v_ref):
    i = pl.program_id(0)
    h = _rms(x_ref[...], ln1_ref[...]).astype(jnp.bfloat16)
    ql = jnp.dot(h, wql_ref[...].astype(jnp.bfloat16),
                 preferred_element_type=jnp.float32)
    qh = jnp.dot(h, wqh_ref[...].astype(jnp.bfloat16),
                 preferred_element_type=jnp.float32)
    kl = jnp.dot(h, wkl_ref[...].astype(jnp.bfloat16),
                 preferred_element_type=jnp.float32)
    kh = jnp.dot(h, wkh_ref[...].astype(jnp.bfloat16),
                 preferred_element_type=jnp.float32)
    v_ref[...] = jnp.dot(h, wv_ref[...].astype(jnp.bfloat16),
                         preferred_element_type=jnp.float32).astype(jnp.bfloat16)
    pos = (i * S_TILE + jax.lax.broadcasted_iota(jnp.int32, (S_TILE, H * DH // 2), 0)
           ).astype(jnp.float32)
    lane = jax.lax.broadcasted_iota(jnp.int32, (S_TILE, H * DH // 2), 1) % (DH // 2)
    inv_freq = jnp.exp(lane.astype(jnp.float32) * (-2.0 * _LN1E4 / DH))
    theta = pos * inv_freq
    c = jnp.cos(theta)
    s = jnp.sin(theta)
    qa_ref[...] = (ql * c - qh * s).astype(jnp.bfloat16)
    qb_ref[...] = (qh * c + ql * s).astype(jnp.bfloat16)
    ka_ref[...] = (kl * c - kh * s).astype(jnp.bfloat16)
    kb_ref[...] = (kh * c + kl * s).astype(jnp.bfloat16)


def _k2_body(qa_ref, qb_ref, ka_ref, kb_ref, v_ref, o_ref):
    qi = pl.program_id(0)
    rowp = qi * S_TILE + jax.lax.broadcasted_iota(jnp.int32, (S_TILE, S_TILE), 0)
    scale = 1.0 / math.sqrt(DH)
    for h in range(H):
        ha = slice(h * (DH // 2), (h + 1) * (DH // 2))
        hv = slice(h * DH, (h + 1) * DH)
        q = jnp.concatenate([qa_ref[:, ha], qb_ref[:, ha]], axis=1)

        def body(j, carry, q=q):
            m, l, acc = carry
            k = jnp.concatenate(
                [ka_ref[pl.ds(j * S_TILE, S_TILE), ha],
                 kb_ref[pl.ds(j * S_TILE, S_TILE), ha]], axis=1)
            v = v_ref[pl.ds(j * S_TILE, S_TILE), hv]
            s = jax.lax.dot_general(q, k, (((1,), (1,)), ((), ())),
                                    preferred_element_type=jnp.float32) * scale
            colp = j * S_TILE + jax.lax.broadcasted_iota(
                jnp.int32, (S_TILE, S_TILE), 1)
            s = jnp.where(colp <= rowp, s, _NEG)
            m_new = jnp.maximum(m, jnp.max(s, axis=1, keepdims=True))
            p = jnp.exp(s - m_new)
            sc = jnp.exp(m - m_new)
            l_new = l * sc + jnp.sum(p, axis=1, keepdims=True)
            acc_new = acc * sc + jnp.dot(p.astype(jnp.bfloat16), v,
                                         preferred_element_type=jnp.float32)
            return m_new, l_new, acc_new

        m0 = jnp.full((S_TILE, 1), _NEG, jnp.float32)
        l0 = jnp.zeros((S_TILE, 1), jnp.float32)
        a0 = jnp.zeros((S_TILE, DH), jnp.float32)
        m, l, acc = jax.lax.fori_loop(0, qi + 1, body, (m0, l0, a0))
        o_ref[:, hv] = (acc / l).astype(jnp.bfloat16)


def _k3_body(ctx_ref, x_ref, wo_ref, ln2_ref, rw_ref, x1_ref, h2_ref, lg_ref):
    x1 = x_ref[...] + jnp.dot(ctx_ref[...], wo_ref[...].astype(jnp.bfloat16),
                              preferred_element_type=jnp.float32)
    x1_ref[...] = x1
    h2 = _rms(x1, ln2_ref[...])
    h2_ref[...] = h2
    lg_ref[...] = jnp.dot(h2, rw_ref[...], preferred_element_type=jnp.float32)


def _k4_body(lg_ref, pos0_ref, pos1_ref, g0_ref, g1_ref, be_ref, aux_ref):
    lg = lg_ref[...]
    mx = jnp.max(lg, axis=1, keepdims=True)
    ex = jnp.exp(lg - mx)
    probs = ex / jnp.sum(ex, axis=1, keepdims=True)
    lane = jax.lax.broadcasted_iota(jnp.int32, (S, E), 1)
    v1 = jnp.max(probs, axis=1, keepdims=True)
    i1 = jnp.min(jnp.where(probs == v1, lane, E), axis=1, keepdims=True)
    oh1 = (lane == i1).astype(jnp.float32)
    masked = jnp.where(lane == i1, _NEG, probs)
    v2 = jnp.max(masked, axis=1, keepdims=True)
    i2 = jnp.min(jnp.where(masked == v2, lane, E), axis=1, keepdims=True)
    oh2 = (lane == i2).astype(jnp.float32)
    tot = v1 + v2
    g0_ref[...] = v1 / tot
    g1_ref[...] = v2 / tot
    m = oh1 + oh2
    lt = (jax.lax.broadcasted_iota(jnp.int32, (S_TILE, S_TILE), 0)
          > jax.lax.broadcasted_iota(jnp.int32, (S_TILE, S_TILE), 1)
          ).astype(jnp.float32)
    rank_chunks = []
    running = jnp.zeros((1, E), jnp.float32)
    for ci in range(NT):
        mc = m[ci * S_TILE:(ci + 1) * S_TILE]
        rank_chunks.append(
            jnp.dot(lt, mc, preferred_element_type=jnp.float32) + running)
        running = running + jnp.sum(mc, axis=0, keepdims=True)
    rank = jnp.concatenate(rank_chunks, axis=0)
    cnt = running
    c_pad = jnp.ceil(cnt * (1.0 / BLK)) * BLK
    up = (jax.lax.broadcasted_iota(jnp.int32, (E, E), 0)
          < jax.lax.broadcasted_iota(jnp.int32, (E, E), 1)).astype(jnp.float32)
    start = jnp.dot(c_pad, up, preferred_element_type=jnp.float32)
    slot = start + rank
    pos0_ref[...] = jnp.sum(oh1 * slot, axis=1, keepdims=True).astype(jnp.int32)
    pos1_ref[...] = jnp.sum(oh2 * slot, axis=1, keepdims=True).astype(jnp.int32)
    bstart = (jax.lax.broadcasted_iota(jnp.int32, (NBLK, E), 0) * BLK
              ).astype(jnp.float32)
    be = jnp.sum((jnp.broadcast_to(start, (NBLK, E)) <= bstart)
                 .astype(jnp.float32), axis=1, keepdims=True) - 1.0
    be = jnp.clip(be, 0.0, E - 1.0)
    total = start[:, E - 1:E] + c_pad[:, E - 1:E]
    bcol = (jax.lax.broadcasted_iota(jnp.int32, (NBLK, 1), 0) * BLK
            ).astype(jnp.float32)
    be_ref[...] = jnp.where(bcol < total, be, -1.0).astype(jnp.int32)
    f = jnp.sum(m, axis=0, keepdims=True) / (S * TOPK)
    pbar = jnp.sum(probs, axis=0, keepdims=True) / S
    aux_ref[...] = E * jnp.sum(f * pbar, axis=1, keepdims=True)


def _k5_sc(h2_hbm, pos0_hbm, pos1_hbm, disp_hbm, rows_v, idx_v, sem):
    c = jax.lax.axis_index("c")
    s = jax.lax.axis_index("s")
    wid = s * 2 + c
    base = wid * CHUNK
    pltpu.sync_copy(h2_hbm.at[pl.ds(base, CHUNK), :], rows_v)
    pltpu.sync_copy(pos0_hbm.at[pl.ds(base, CHUNK)], idx_v)
    pltpu.async_copy(rows_v, disp_hbm.at[idx_v], sem).wait()
    pltpu.sync_copy(pos1_hbm.at[pl.ds(base, CHUNK)], idx_v)
    pltpu.async_copy(rows_v, disp_hbm.at[idx_v], sem).wait()


def _k6_body(be_ref, disp_ref, w1_ref, w2_ref, y_ref):
    e = be_ref[pl.program_id(0)]

    @pl.when(e >= 0)
    def _():
        t = disp_ref[...].astype(jnp.bfloat16)
        a = jnp.dot(t, w1_ref[0].astype(jnp.bfloat16),
                    preferred_element_type=jnp.float32)
        g = jax.nn.gelu(a)
        y_ref[...] = jnp.dot(g.astype(jnp.bfloat16),
                             w2_ref[0].astype(jnp.bfloat16),
                             preferred_element_type=jnp.float32)


def _k7_sc(y_hbm, pos0_hbm, pos1_hbm, y0_hbm, y1_hbm, rows_v, idx_v, sem):
    c = jax.lax.axis_index("c")
    s = jax.lax.axis_index("s")
    wid = s * 2 + c
    base = wid * CHUNK
    pltpu.sync_copy(pos0_hbm.at[pl.ds(base, CHUNK)], idx_v)
    pltpu.async_copy(y_hbm.at[idx_v], rows_v, sem).wait()
    pltpu.sync_copy(rows_v, y0_hbm.at[pl.ds(base, CHUNK), :])
    pltpu.sync_copy(pos1_hbm.at[pl.ds(base, CHUNK)], idx_v)
    pltpu.async_copy(y_hbm.at[idx_v], rows_v, sem).wait()
    pltpu.sync_copy(rows_v, y1_hbm.at[pl.ds(base, CHUNK), :])


def _k8_body(x1_ref, y0_ref, y1_ref, g0_ref, g1_ref, out_ref):
    out_ref[...] = (x1_ref[...] + g0_ref[...] * y0_ref[...]
                    + g1_ref[...] * y1_ref[...])


_SC_MESH = dict(core_axis_name="c", subcore_axis_name="s")


def _dispatch(h2, pos0f, pos1f):
    mesh = plsc.VectorSubcoreMesh(**_SC_MESH)
    return pl.kernel(
        _k5_sc,
        mesh=mesh,
        out_type=jax.ShapeDtypeStruct((CAP, D), jnp.float32),
        scratch_types=[pltpu.VMEM((CHUNK, D), jnp.float32),
                       pltpu.VMEM((CHUNK,), jnp.int32),
                       pltpu.SemaphoreType.DMA],
    )(h2, pos0f, pos1f)


def _undispatch(y, pos0f, pos1f):
    mesh = plsc.VectorSubcoreMesh(**_SC_MESH)
    return pl.kernel(
        _k7_sc,
        mesh=mesh,
        out_type=[jax.ShapeDtypeStruct((S, D), jnp.float32),
                  jax.ShapeDtypeStruct((S, D), jnp.float32)],
        scratch_types=[pltpu.VMEM((CHUNK, D), jnp.float32),
                       pltpu.VMEM((CHUNK,), jnp.int32),
                       pltpu.SemaphoreType.DMA],
    )(y, pos0f, pos1f)


def kernel(x, ln1_w, ln2_w, wq, wk, wv, wo, router_w, w1, w2):
    xs = x.reshape(S, D)
    ln1 = ln1_w.reshape(1, D)
    ln2 = ln2_w.reshape(1, D)
    wq4 = wq.reshape(D, H, 2, DH // 2).transpose(0, 2, 1, 3).reshape(D, 2, D // 2)
    wk4 = wk.reshape(D, H, 2, DH // 2).transpose(0, 2, 1, 3).reshape(D, 2, D // 2)
    wql, wqh = wq4[:, 0], wq4[:, 1]
    wkl, wkh = wk4[:, 0], wk4[:, 1]

    full = lambda shape: pl.BlockSpec(shape, lambda *idx: tuple(0 for _ in shape))
    row_tile = lambda w, tile: pl.BlockSpec((tile, w), lambda i, *_: (i, 0))

    qa, qb, ka, kb, v = pl.pallas_call(
        _k1_body,
        grid=(NT,),
        in_specs=[row_tile(D, S_TILE), full((1, D)), full((D, D // 2)),
                  full((D, D // 2)), full((D, D // 2)), full((D, D // 2)),
                  full((D, D))],
        out_specs=[row_tile(D // 2, S_TILE)] * 4 + [row_tile(D, S_TILE)],
        out_shape=[jax.ShapeDtypeStruct((S, D // 2), jnp.bfloat16)] * 4
        + [jax.ShapeDtypeStruct((S, D), jnp.bfloat16)],
    )(xs, ln1, wql, wqh, wkl, wkh, wv)

    ctx = pl.pallas_call(
        _k2_body,
        grid=(NT,),
        in_specs=[row_tile(D // 2, S_TILE), row_tile(D // 2, S_TILE),
                  full((S, D // 2)), full((S, D // 2)), full((S, D))],
        out_specs=row_tile(D, S_TILE),
        out_shape=jax.ShapeDtypeStruct((S, D), jnp.bfloat16),
    )(qa, qb, ka, kb, v)

    x1, h2, logits = pl.pallas_call(
        _k3_body,
        grid=(NT,),
        in_specs=[row_tile(D, S_TILE), row_tile(D, S_TILE), full((D, D)),
                  full((1, D)), full((D, E))],
        out_specs=[row_tile(D, S_TILE), row_tile(D, S_TILE), row_tile(E, S_TILE)],
        out_shape=[jax.ShapeDtypeStruct((S, D), jnp.float32),
                   jax.ShapeDtypeStruct((S, D), jnp.float32),
                   jax.ShapeDtypeStruct((S, E), jnp.float32)],
    )(ctx, xs, wo, ln2, router_w)

    pos0, pos1, g0, g1, be, aux = pl.pallas_call(
        _k4_body,
        grid=(1,),
        in_specs=[full((S, E))],
        out_specs=[full((S, 1)), full((S, 1)), full((S, 1)), full((S, 1)),
                   full((NBLK, 1)), full((1, 1))],
        out_shape=[jax.ShapeDtypeStruct((S, 1), jnp.int32),
                   jax.ShapeDtypeStruct((S, 1), jnp.int32),
                   jax.ShapeDtypeStruct((S, 1), jnp.float32),
                   jax.ShapeDtypeStruct((S, 1), jnp.float32),
                   jax.ShapeDtypeStruct((NBLK, 1), jnp.int32),
                   jax.ShapeDtypeStruct((1, 1), jnp.float32)],
    )(logits)

    pos0f = pos0.reshape(S)
    pos1f = pos1.reshape(S)

    disp = _dispatch(h2, pos0f, pos1f)

    y = pl.pallas_call(
        _k6_body,
        grid_spec=pltpu.PrefetchScalarGridSpec(
            num_scalar_prefetch=1,
            grid=(NBLK,),
            in_specs=[
                pl.BlockSpec((BLK, D), lambda b, be_s: (b, 0)),
                pl.BlockSpec((1, D, FF),
                             lambda b, be_s: (jnp.maximum(be_s[b], 0), 0, 0)),
                pl.BlockSpec((1, FF, D),
                             lambda b, be_s: (jnp.maximum(be_s[b], 0), 0, 0)),
            ],
            out_specs=pl.BlockSpec((BLK, D), lambda b, be_s: (b, 0)),
        ),
        out_shape=jax.ShapeDtypeStruct((CAP, D), jnp.float32),
    )(be.reshape(NBLK), disp, w1, w2)

    y0, y1 = _undispatch(y, pos0f, pos1f)

    x2 = pl.pallas_call(
        _k8_body,
        grid=(NT,),
        in_specs=[row_tile(D, S_TILE), row_tile(D, S_TILE), row_tile(D, S_TILE),
                  row_tile(1, S_TILE), row_tile(1, S_TILE)],
        out_specs=row_tile(D, S_TILE),
        out_shape=jax.ShapeDtypeStruct((S, D), jnp.float32),
    )(x1, y0, y1, g0, g1)

    return x2.reshape(B, S, D), aux.reshape(())


# speedup vs baseline: 1.5477x; 1.5091x over previous
import functools
import math

import jax
import jax.numpy as jnp
from jax.experimental import pallas as pl
from jax.experimental.pallas import tpu as pltpu
from jax.experimental.pallas import tpu_sc as plsc

B, S, D = 1, 2048, 1024
H, DH = 16, 64
E, TOPK, FF = 8, 2, 2048
EPS = 1e-6

S_TILE = 256
NT = S // S_TILE
BLK = 128
CAP = S * TOPK + E * BLK - E * 1
CAP = 5120
NBLK = CAP // BLK

NW = 32
CHUNK = S // NW

_NEG = -1e30
_LN1E4 = math.log(10000.0)


def _rms(h, w):
    var = jnp.mean(h * h, axis=-1, keepdims=True)
    return h * jax.lax.rsqrt(var + EPS) * w


def _k1_body(x_ref, ln1_ref, wql_ref, wqh_ref, wkl_ref, wkh_ref, wv_ref,
             qa_ref, qb_ref, ka_ref, kb_ref, v_ref):
    i = pl.program_id(0)
    h = _rms(x_ref[...], ln1_ref[...]).astype(jnp.bfloat16)
    ql = jnp.dot(h, wql_ref[...].astype(jnp.bfloat16),
                 preferred_element_type=jnp.float32)
    qh = jnp.dot(h, wqh_ref[...].astype(jnp.bfloat16),
                 preferred_element_type=jnp.float32)
    kl = jnp.dot(h, wkl_ref[...].astype(jnp.bfloat16),
                 preferred_element_type=jnp.float32)
    kh = jnp.dot(h, wkh_ref[...].astype(jnp.bfloat16),
                 preferred_element_type=jnp.float32)
    v_ref[...] = jnp.dot(h, wv_ref[...].astype(jnp.bfloat16),
                         preferred_element_type=jnp.float32).astype(jnp.bfloat16)
    pos = (i * S_TILE + jax.lax.broadcasted_iota(jnp.int32, (S_TILE, H * DH // 2), 0)
           ).astype(jnp.float32)
    lane = jax.lax.broadcasted_iota(jnp.int32, (S_TILE, H * DH // 2), 1) % (DH // 2)
    inv_freq = jnp.exp(lane.astype(jnp.float32) * (-2.0 * _LN1E4 / DH))
    theta = pos * inv_freq
    c = jnp.cos(theta)
    s = jnp.sin(theta)
    qa_ref[...] = (ql * c - qh * s).astype(jnp.bfloat16)
    qb_ref[...] = (qh * c + ql * s).astype(jnp.bfloat16)
    ka_ref[...] = (kl * c - kh * s).astype(jnp.bfloat16)
    kb_ref[...] = (kh * c + kl * s).astype(jnp.bfloat16)


def _k2_body(qa_ref, qb_ref, ka_ref, kb_ref, v_ref, o_ref):
    qi = pl.program_id(0)
    scale = 1.0 / math.sqrt(DH)
    rowp = qi * S_TILE + jax.lax.broadcasted_iota(jnp.int32, (S_TILE, S), 0)
    colp = jax.lax.broadcasted_iota(jnp.int32, (S_TILE, S), 1)
    mask = colp <= rowp
    for h in range(H):
        ha = slice(h * (DH // 2), (h + 1) * (DH // 2))
        hv = slice(h * DH, (h + 1) * DH)
        q = jnp.concatenate([qa_ref[:, ha], qb_ref[:, ha]], axis=1)
        k = jnp.concatenate([ka_ref[:, ha], kb_ref[:, ha]], axis=1)
        s = jax.lax.dot_general(q, k, (((1,), (1,)), ((), ())),
                                preferred_element_type=jnp.float32) * scale
        s = jnp.where(mask, s, _NEG)
        mx = jnp.max(s, axis=1, keepdims=True)
        p = jnp.exp(s - mx)
        l = jnp.sum(p, axis=1, keepdims=True)
        ctx = jnp.dot(p.astype(jnp.bfloat16), v_ref[:, hv],
                      preferred_element_type=jnp.float32)
        o_ref[:, hv] = (ctx / l).astype(jnp.bfloat16)


def _k3_body(ctx_ref, x_ref, wo_ref, ln2_ref, rw_ref, x1_ref, h2_ref, lg_ref):
    x1 = x_ref[...] + jnp.dot(ctx_ref[...], wo_ref[...].astype(jnp.bfloat16),
                              preferred_element_type=jnp.float32)
    x1_ref[...] = x1
    h2 = _rms(x1, ln2_ref[...])
    h2_ref[...] = h2
    lg_ref[...] = jnp.dot(h2, rw_ref[...], preferred_element_type=jnp.float32)


def _k4_body(lg_ref, pos0_ref, pos1_ref, g0_ref, g1_ref, be_ref, aux_ref):
    lg = lg_ref[...]
    mx = jnp.max(lg, axis=1, keepdims=True)
    ex = jnp.exp(lg - mx)
    probs = ex / jnp.sum(ex, axis=1, keepdims=True)
    lane = jax.lax.broadcasted_iota(jnp.int32, (S, E), 1)
    v1 = jnp.max(probs, axis=1, keepdims=True)
    i1 = jnp.min(jnp.where(probs == v1, lane, E), axis=1, keepdims=True)
    oh1 = (lane == i1).astype(jnp.float32)
    masked = jnp.where(lane == i1, _NEG, probs)
    v2 = jnp.max(masked, axis=1, keepdims=True)
    i2 = jnp.min(jnp.where(masked == v2, lane, E), axis=1, keepdims=True)
    oh2 = (lane == i2).astype(jnp.float32)
    tot = v1 + v2
    g0_ref[...] = v1 / tot
    g1_ref[...] = v2 / tot
    m = oh1 + oh2
    lt = (jax.lax.broadcasted_iota(jnp.int32, (S_TILE, S_TILE), 0)
          > jax.lax.broadcasted_iota(jnp.int32, (S_TILE, S_TILE), 1)
          ).astype(jnp.float32)
    rank_chunks = []
    running = jnp.zeros((1, E), jnp.float32)
    for ci in range(NT):
        mc = m[ci * S_TILE:(ci + 1) * S_TILE]
        rank_chunks.append(
            jnp.dot(lt, mc, preferred_element_type=jnp.float32) + running)
        running = running + jnp.sum(mc, axis=0, keepdims=True)
    rank = jnp.concatenate(rank_chunks, axis=0)
    cnt = running
    c_pad = jnp.ceil(cnt * (1.0 / BLK)) * BLK
    up = (jax.lax.broadcasted_iota(jnp.int32, (E, E), 0)
          < jax.lax.broadcasted_iota(jnp.int32, (E, E), 1)).astype(jnp.float32)
    start = jnp.dot(c_pad, up, preferred_element_type=jnp.float32)
    slot = start + rank
    pos0_ref[...] = jnp.sum(oh1 * slot, axis=1, keepdims=True).astype(jnp.int32)
    pos1_ref[...] = jnp.sum(oh2 * slot, axis=1, keepdims=True).astype(jnp.int32)
    bstart = (jax.lax.broadcasted_iota(jnp.int32, (NBLK, E), 0) * BLK
              ).astype(jnp.float32)
    be = jnp.sum((jnp.broadcast_to(start, (NBLK, E)) <= bstart)
                 .astype(jnp.float32), axis=1, keepdims=True) - 1.0
    be = jnp.clip(be, 0.0, E - 1.0)
    total = start[:, E - 1:E] + c_pad[:, E - 1:E]
    bcol = (jax.lax.broadcasted_iota(jnp.int32, (NBLK, 1), 0) * BLK
            ).astype(jnp.float32)
    be_ref[...] = jnp.where(bcol < total, be, -1.0).astype(jnp.int32)
    f = jnp.sum(m, axis=0, keepdims=True) / (S * TOPK)
    pbar = jnp.sum(probs, axis=0, keepdims=True) / S
    aux_ref[...] = E * jnp.sum(f * pbar, axis=1, keepdims=True)


def _k5_sc(h2_hbm, pos0_hbm, pos1_hbm, disp_hbm, rows_v, idx_v, sem):
    c = jax.lax.axis_index("c")
    s = jax.lax.axis_index("s")
    wid = s * 2 + c
    base = wid * CHUNK
    pltpu.sync_copy(h2_hbm.at[pl.ds(base, CHUNK), :], rows_v)
    pltpu.sync_copy(pos0_hbm.at[pl.ds(base, CHUNK)], idx_v)
    pltpu.async_copy(rows_v, disp_hbm.at[idx_v], sem).wait()
    pltpu.sync_copy(pos1_hbm.at[pl.ds(base, CHUNK)], idx_v)
    pltpu.async_copy(rows_v, disp_hbm.at[idx_v], sem).wait()


def _k6_body(be_ref, disp_ref, w1_ref, w2_ref, y_ref):
    e = be_ref[pl.program_id(0)]

    @pl.when(e >= 0)
    def _():
        t = disp_ref[...].astype(jnp.bfloat16)
        a = jnp.dot(t, w1_ref[0].astype(jnp.bfloat16),
                    preferred_element_type=jnp.float32)
        g = jax.nn.gelu(a)
        y_ref[...] = jnp.dot(g.astype(jnp.bfloat16),
                             w2_ref[0].astype(jnp.bfloat16),
                             preferred_element_type=jnp.float32)


def _k7_sc(y_hbm, pos0_hbm, pos1_hbm, y0_hbm, y1_hbm, rows_v, idx_v, sem):
    c = jax.lax.axis_index("c")
    s = jax.lax.axis_index("s")
    wid = s * 2 + c
    base = wid * CHUNK
    pltpu.sync_copy(pos0_hbm.at[pl.ds(base, CHUNK)], idx_v)
    pltpu.async_copy(y_hbm.at[idx_v], rows_v, sem).wait()
    pltpu.sync_copy(rows_v, y0_hbm.at[pl.ds(base, CHUNK), :])
    pltpu.sync_copy(pos1_hbm.at[pl.ds(base, CHUNK)], idx_v)
    pltpu.async_copy(y_hbm.at[idx_v], rows_v, sem).wait()
    pltpu.sync_copy(rows_v, y1_hbm.at[pl.ds(base, CHUNK), :])


def _k8_body(x1_ref, y0_ref, y1_ref, g0_ref, g1_ref, out_ref):
    out_ref[...] = (x1_ref[...] + g0_ref[...] * y0_ref[...]
                    + g1_ref[...] * y1_ref[...])


_SC_MESH = dict(core_axis_name="c", subcore_axis_name="s")


def _dispatch(h2, pos0f, pos1f):
    mesh = plsc.VectorSubcoreMesh(**_SC_MESH)
    return pl.kernel(
        _k5_sc,
        mesh=mesh,
        out_type=jax.ShapeDtypeStruct((CAP, D), jnp.float32),
        scratch_types=[pltpu.VMEM((CHUNK, D), jnp.float32),
                       pltpu.VMEM((CHUNK,), jnp.int32),
                       pltpu.SemaphoreType.DMA],
    )(h2, pos0f, pos1f)


def _undispatch(y, pos0f, pos1f):
    mesh = plsc.VectorSubcoreMesh(**_SC_MESH)
    return pl.kernel(
        _k7_sc,
        mesh=mesh,
        out_type=[jax.ShapeDtypeStruct((S, D), jnp.float32),
                  jax.ShapeDtypeStruct((S, D), jnp.float32)],
        scratch_types=[pltpu.VMEM((CHUNK, D), jnp.float32),
                       pltpu.VMEM((CHUNK,), jnp.int32),
                       pltpu.SemaphoreType.DMA],
    )(y, pos0f, pos1f)


def kernel(x, ln1_w, ln2_w, wq, wk, wv, wo, router_w, w1, w2):
    xs = x.reshape(S, D)
    ln1 = ln1_w.reshape(1, D)
    ln2 = ln2_w.reshape(1, D)
    wq4 = wq.reshape(D, H, 2, DH // 2).transpose(0, 2, 1, 3).reshape(D, 2, D // 2)
    wk4 = wk.reshape(D, H, 2, DH // 2).transpose(0, 2, 1, 3).reshape(D, 2, D // 2)
    wql, wqh = wq4[:, 0], wq4[:, 1]
    wkl, wkh = wk4[:, 0], wk4[:, 1]

    full = lambda shape: pl.BlockSpec(shape, lambda *idx: tuple(0 for _ in shape))
    row_tile = lambda w, tile: pl.BlockSpec((tile, w), lambda i, *_: (i, 0))

    qa, qb, ka, kb, v = pl.pallas_call(
        _k1_body,
        grid=(NT,),
        in_specs=[row_tile(D, S_TILE), full((1, D)), full((D, D // 2)),
                  full((D, D // 2)), full((D, D // 2)), full((D, D // 2)),
                  full((D, D))],
        out_specs=[row_tile(D // 2, S_TILE)] * 4 + [row_tile(D, S_TILE)],
        out_shape=[jax.ShapeDtypeStruct((S, D // 2), jnp.bfloat16)] * 4
        + [jax.ShapeDtypeStruct((S, D), jnp.bfloat16)],
    )(xs, ln1, wql, wqh, wkl, wkh, wv)

    ctx = pl.pallas_call(
        _k2_body,
        grid=(NT,),
        in_specs=[row_tile(D // 2, S_TILE), row_tile(D // 2, S_TILE),
                  full((S, D // 2)), full((S, D // 2)), full((S, D))],
        out_specs=row_tile(D, S_TILE),
        out_shape=jax.ShapeDtypeStruct((S, D), jnp.bfloat16),
    )(qa, qb, ka, kb, v)

    x1, h2, logits = pl.pallas_call(
        _k3_body,
        grid=(NT,),
        in_specs=[row_tile(D, S_TILE), row_tile(D, S_TILE), full((D, D)),
                  full((1, D)), full((D, E))],
        out_specs=[row_tile(D, S_TILE), row_tile(D, S_TILE), row_tile(E, S_TILE)],
        out_shape=[jax.ShapeDtypeStruct((S, D), jnp.float32),
                   jax.ShapeDtypeStruct((S, D), jnp.float32),
                   jax.ShapeDtypeStruct((S, E), jnp.float32)],
    )(ctx, xs, wo, ln2, router_w)

    pos0, pos1, g0, g1, be, aux = pl.pallas_call(
        _k4_body,
        grid=(1,),
        in_specs=[full((S, E))],
        out_specs=[full((S, 1)), full((S, 1)), full((S, 1)), full((S, 1)),
                   full((NBLK, 1)), full((1, 1))],
        out_shape=[jax.ShapeDtypeStruct((S, 1), jnp.int32),
                   jax.ShapeDtypeStruct((S, 1), jnp.int32),
                   jax.ShapeDtypeStruct((S, 1), jnp.float32),
                   jax.ShapeDtypeStruct((S, 1), jnp.float32),
                   jax.ShapeDtypeStruct((NBLK, 1), jnp.int32),
                   jax.ShapeDtypeStruct((1, 1), jnp.float32)],
    )(logits)

    pos0f = pos0.reshape(S)
    pos1f = pos1.reshape(S)

    disp = _dispatch(h2, pos0f, pos1f)

    y = pl.pallas_call(
        _k6_body,
        grid_spec=pltpu.PrefetchScalarGridSpec(
            num_scalar_prefetch=1,
            grid=(NBLK,),
            in_specs=[
                pl.BlockSpec((BLK, D), lambda b, be_s: (b, 0)),
                pl.BlockSpec((1, D, FF),
                             lambda b, be_s: (jnp.maximum(be_s[b], 0), 0, 0)),
                pl.BlockSpec((1, FF, D),
                             lambda b, be_s: (jnp.maximum(be_s[b], 0), 0, 0)),
            ],
            out_specs=pl.BlockSpec((BLK, D), lambda b, be_s: (b, 0)),
        ),
        out_shape=jax.ShapeDtypeStruct((CAP, D), jnp.float32),
    )(be.reshape(NBLK), disp, w1, w2)

    y0, y1 = _undispatch(y, pos0f, pos1f)

    x2 = pl.pallas_call(
        _k8_body,
        grid=(NT,),
        in_specs=[row_tile(D, S_TILE), row_tile(D, S_TILE), row_tile(D, S_TILE),
                  row_tile(1, S_TILE), row_tile(1, S_TILE)],
        out_specs=row_tile(D, S_TILE),
        out_shape=jax.ShapeDtypeStruct((S, D), jnp.float32),
    )(x1, y0, y1, g0, g1)

    return x2.reshape(B, S, D), aux.reshape(())
